# Initial kernel scaffold; baseline (speedup 1.0000x reference)
import jax
import jax.numpy as jnp
from jax.experimental import pallas as pl


def kernel(user_emb, item_emb, adj_indices, adj_values):
    raise NotImplementedError("write your pallas kernel here")


# speedup vs baseline: 2.6955x; 2.6955x over previous
import functools

import jax
import jax.numpy as jnp
from jax import lax
from jax.experimental import pallas as pl
from jax.experimental.pallas import tpu as pltpu
from jax.experimental.pallas import tpu_sc as plsc

_USER_NUM = 5000
_ITEM_NUM = 5000
_N_NODES = _USER_NUM + _ITEM_NUM
_EMB = 128
_N_EDGES = 320000

_NP = 10240
_EP = 327680
_CHUNKS = 80
_CHUNK = 128
_ROWS_PER_TILE = _NP // 16
_VECS = _EMB // 16


def _spmm_body(ego, srcs, dsts, vals, out, accum, src_all, dst_all, val_all,
               rows, sem):
    c_id = lax.axis_index("c")
    s_id = lax.axis_index("s")
    wid = c_id * 16 + s_id

    zf = jnp.zeros((16,), jnp.float32)

    def zrow(e, carry):
        for j in range(_VECS):
            rows[e, pl.ds(j * 16, 16)] = zf
        return carry

    lax.fori_loop(0, _CHUNK, zrow, 0)
    for i in range(_ROWS_PER_TILE // _CHUNK):
        pltpu.sync_copy(rows, accum.at[pl.ds(s_id * _ROWS_PER_TILE + i * _CHUNK,
                                             _CHUNK)])

    pltpu.sync_copy(srcs.at[wid], src_all)
    pltpu.sync_copy(dsts.at[wid], dst_all)
    pltpu.sync_copy(vals.at[wid], val_all)
    plsc.subcore_barrier()

    def chunk(c, carry):
        pltpu.async_copy(ego.at[src_all.at[c]], rows, sem).wait()

        def grpfn(g, carry2):
            vvec = val_all[pl.ds(c * _CHUNK + g * 16, 16)]
            for l in range(16):
                e = g * 16 + l
                s = vvec[l]
                for j in range(_VECS):
                    sl = pl.ds(j * 16, 16)
                    rows[e, sl] = rows[e, sl] * s
            return carry2

        lax.fori_loop(0, _CHUNK // 16, grpfn, 0)
        pltpu.sync_copy(rows, accum.at[dst_all.at[c]], add=True)
        return carry

    lax.fori_loop(0, _CHUNKS, chunk, 0)
    plsc.subcore_barrier()

    for i in range(_ROWS_PER_TILE // _CHUNK):
        r0 = s_id * _ROWS_PER_TILE + i * _CHUNK
        pltpu.sync_copy(accum.at[pl.ds(r0, _CHUNK)],
                        out.at[c_id, pl.ds(r0, _CHUNK)])


@functools.cache
def _get_spmm():
    mesh = plsc.VectorSubcoreMesh(core_axis_name="c", subcore_axis_name="s",
                                  num_cores=2, num_subcores=16)
    return pl.kernel(
        _spmm_body,
        out_type=jax.ShapeDtypeStruct((2, _NP, _EMB), jnp.float32),
        mesh=mesh,
        scratch_types=[
            pltpu.VMEM_SHARED((_NP, _EMB), jnp.float32),
            pltpu.VMEM((_CHUNKS, _CHUNK), jnp.int32),
            pltpu.VMEM((_CHUNKS, _CHUNK), jnp.int32),
            pltpu.VMEM((_CHUNKS * _CHUNK,), jnp.float32),
            pltpu.VMEM((_CHUNK, _EMB), jnp.float32),
            pltpu.SemaphoreType.DMA,
        ],
    )


_BLK = 256


def _add2_body(a, b, o):
    o[...] = a[...] + b[...]


def _final_body(e1, e2, p0, p1, o):
    o[...] = (e1[...] + e2[...] + p0[...] + p1[...]) * jnp.float32(1.0 / 3.0)


def _tc_call(body, n_in):
    spec = pl.BlockSpec((_BLK, _EMB), lambda i: (i, 0))
    return pl.pallas_call(
        body,
        grid=(_NP // _BLK,),
        in_specs=[spec] * n_in,
        out_specs=spec,
        out_shape=jax.ShapeDtypeStruct((_NP, _EMB), jnp.float32),
    )


def kernel(user_emb, item_emb, adj_indices, adj_values):
    ego0 = jnp.concatenate([user_emb, item_emb], axis=0)
    ego0 = jnp.pad(ego0, ((0, _NP - _N_NODES), (0, 0)))

    pad = _EP - _N_EDGES
    src = jnp.concatenate(
        [adj_indices[0], jnp.zeros((pad,), jnp.int32)]).reshape(32, _CHUNKS,
                                                                _CHUNK)
    dst = jnp.concatenate(
        [adj_indices[1], jnp.full((pad,), _NP - 1, jnp.int32)]).reshape(
            32, _CHUNKS, _CHUNK)
    val = jnp.concatenate(
        [adj_values, jnp.zeros((pad,), jnp.float32)]).reshape(
            32, _CHUNKS * _CHUNK)

    spmm = _get_spmm()
    p1 = spmm(ego0, src, dst, val)
    e1 = _tc_call(_add2_body, 2)(p1[0], p1[1])
    p2 = spmm(e1, src, dst, val)
    e2 = _tc_call(_add2_body, 2)(p2[0], p2[1])
    p3 = spmm(e2, src, dst, val)
    mean = _tc_call(_final_body, 4)(e1, e2, p3[0], p3[1])

    return mean[:_USER_NUM], mean[_USER_NUM:_N_NODES]


# speedup vs baseline: 3.1042x; 1.1516x over previous
import functools

import jax
import jax.numpy as jnp
from jax import lax
from jax.experimental import pallas as pl
from jax.experimental.pallas import tpu as pltpu
from jax.experimental.pallas import tpu_sc as plsc

_USER_NUM = 5000
_ITEM_NUM = 5000
_N_NODES = _USER_NUM + _ITEM_NUM
_EMB = 128
_N_EDGES = 320000

_NP = 10240
_EP = 327680
_CHUNKS = 80
_CHUNK = 128
_ROWS_PER_TILE = _NP // 16
_VECS = _EMB // 16


def _spmm_body(ego, edges, vals, out, accum, idx4, val4, rows_a, rows_b,
               isem0, isem1, isem2, isem3, gsem_a, gsem_b):
    c_id = lax.axis_index("c")
    s_id = lax.axis_index("s")
    wid = c_id * 16 + s_id
    rows2 = (rows_a, rows_b)
    gsem2 = (gsem_a, gsem_b)
    isems = (isem0, isem1, isem2, isem3)

    zf = jnp.zeros((16,), jnp.float32)

    def zrow(e, carry):
        for j in range(_VECS):
            rows_a[e, pl.ds(j * 16, 16)] = zf
        return carry

    lax.fori_loop(0, _CHUNK, zrow, 0)
    for i in range(_ROWS_PER_TILE // _CHUNK):
        pltpu.sync_copy(rows_a,
                        accum.at[pl.ds(s_id * _ROWS_PER_TILE + i * _CHUNK,
                                       _CHUNK)])

    for j in range(4):
        pltpu.async_copy(edges.at[wid, j], idx4.at[j], isems[j])
        pltpu.async_copy(vals.at[wid, j], val4.at[j], isems[j])
    for j in range(2):
        pltpu.make_async_copy(edges.at[wid, j], idx4.at[j], isems[j]).wait()
        pltpu.make_async_copy(vals.at[wid, j], val4.at[j], isems[j]).wait()
        pltpu.async_copy(ego.at[idx4.at[j, 0]], rows2[j], gsem2[j])
    plsc.subcore_barrier()

    def do_chunk(c, rows, gsem, idx_cur, val_cur, idx_g2, val_g2, isem_g2,
                 isem_cur):
        pltpu.make_async_copy(ego.at[idx_cur.at[0]], rows, gsem).wait()

        def grpfn(g, carry2):
            vvec = val_cur[pl.ds(g * 16, 16)]
            for l in range(16):
                e = g * 16 + l
                s = vvec[l]
                for j in range(_VECS):
                    sl = pl.ds(j * 16, 16)
                    rows[e, sl] = rows[e, sl] * s
            return carry2

        lax.fori_loop(0, _CHUNK // 16, grpfn, 0)
        pltpu.sync_copy(rows, accum.at[idx_cur.at[1]], add=True)
        pltpu.make_async_copy(edges.at[wid, 0], idx_g2, isem_g2).wait()
        pltpu.make_async_copy(vals.at[wid, 0], val_g2, isem_g2).wait()
        pltpu.async_copy(ego.at[idx_g2.at[0]], rows, gsem)
        cc = jnp.minimum(c + 4, _CHUNKS - 1)
        pltpu.async_copy(edges.at[wid, cc], idx_cur, isem_cur)
        pltpu.async_copy(vals.at[wid, cc], val_cur, isem_cur)

    def superblock(g, carry):
        for i in range(4):
            do_chunk(g * 4 + i, rows2[i % 2], gsem2[i % 2], idx4.at[i],
                     val4.at[i], idx4.at[(i + 2) % 4], val4.at[(i + 2) % 4],
                     isems[(i + 2) % 4], isems[i])
        return carry

    lax.fori_loop(0, _CHUNKS // 4, superblock, 0)
    for j in range(2):
        pltpu.make_async_copy(ego.at[idx4.at[j, 0]], rows2[j], gsem2[j]).wait()
        pltpu.make_async_copy(edges.at[wid, 0], idx4.at[j + 2],
                              isems[j + 2]).wait()
        pltpu.make_async_copy(vals.at[wid, 0], val4.at[j + 2],
                              isems[j + 2]).wait()
    plsc.subcore_barrier()

    for i in range(_ROWS_PER_TILE // _CHUNK):
        r0 = s_id * _ROWS_PER_TILE + i * _CHUNK
        pltpu.sync_copy(accum.at[pl.ds(r0, _CHUNK)],
                        out.at[c_id, pl.ds(r0, _CHUNK)])


@functools.cache
def _get_spmm():
    mesh = plsc.VectorSubcoreMesh(core_axis_name="c", subcore_axis_name="s",
                                  num_cores=2, num_subcores=16)
    return pl.kernel(
        _spmm_body,
        out_type=jax.ShapeDtypeStruct((2, _NP, _EMB), jnp.float32),
        mesh=mesh,
        scratch_types=[
            pltpu.VMEM_SHARED((_NP, _EMB), jnp.float32),
            pltpu.VMEM((4, 2, _CHUNK), jnp.int32),
            pltpu.VMEM((4, _CHUNK), jnp.float32),
            pltpu.VMEM((_CHUNK, _EMB), jnp.float32),
            pltpu.VMEM((_CHUNK, _EMB), jnp.float32),
            pltpu.SemaphoreType.DMA,
            pltpu.SemaphoreType.DMA,
            pltpu.SemaphoreType.DMA,
            pltpu.SemaphoreType.DMA,
            pltpu.SemaphoreType.DMA,
            pltpu.SemaphoreType.DMA,
        ],
    )


_BLK = 256


def _add2_body(a, b, o):
    o[...] = a[...] + b[...]


def _final_body(e1, e2, p0, p1, o):
    o[...] = (e1[...] + e2[...] + p0[...] + p1[...]) * jnp.float32(1.0 / 3.0)


def _tc_call(body, n_in):
    spec = pl.BlockSpec((_BLK, _EMB), lambda i: (i, 0))
    return pl.pallas_call(
        body,
        grid=(_NP // _BLK,),
        in_specs=[spec] * n_in,
        out_specs=spec,
        out_shape=jax.ShapeDtypeStruct((_NP, _EMB), jnp.float32),
    )


def kernel(user_emb, item_emb, adj_indices, adj_values):
    ego0 = jnp.concatenate([user_emb, item_emb], axis=0)
    ego0 = jnp.pad(ego0, ((0, _NP - _N_NODES), (0, 0)))

    pad = _EP - _N_EDGES
    src = jnp.concatenate(
        [adj_indices[0], jnp.zeros((pad,), jnp.int32)]).reshape(32, _CHUNKS,
                                                                _CHUNK)
    dst = jnp.concatenate(
        [adj_indices[1], jnp.full((pad,), _NP - 1, jnp.int32)]).reshape(
            32, _CHUNKS, _CHUNK)
    val = jnp.concatenate(
        [adj_values, jnp.zeros((pad,), jnp.float32)]).reshape(32, _CHUNKS,
                                                              _CHUNK)
    edges = jnp.stack([src, dst], axis=2)

    spmm = _get_spmm()
    p1 = spmm(ego0, edges, val)
    e1 = _tc_call(_add2_body, 2)(p1[0], p1[1])
    p2 = spmm(e1, edges, val)
    e2 = _tc_call(_add2_body, 2)(p2[0], p2[1])
    p3 = spmm(e2, edges, val)
    mean = _tc_call(_final_body, 4)(e1, e2, p3[0], p3[1])

    return mean[:_USER_NUM], mean[_USER_NUM:_N_NODES]
